# baseline (speedup 1.0000x reference)
import functools

import jax
import jax.numpy as jnp
from jax import lax
from jax.experimental import pallas as pl
from jax.experimental.pallas import tpu as pltpu
from jax.experimental.pallas import tpu_sc as plsc

N_CLS = 101
CTX_LEN = 77
CTX_DIM = 512
B = N_CLS * CTX_LEN
NUM_CORES = 2
NUM_SUBCORES = 16
NW = NUM_CORES * NUM_SUBCORES
SUB0 = 128
SUB1 = 120
CHUNK = SUB0 + SUB1
TAIL = B - (NW - 1) * CHUNK
IDX_PAD = (NW - 1) * CHUNK + SUB0


def kernel(tokenized_prompts, token_embedding):
    idx = tokenized_prompts.reshape(-1)
    idx = jnp.concatenate([idx, jnp.zeros((IDX_PAD - B,), jnp.int32)])
    table = token_embedding.reshape(-1, 1, CTX_DIM)

    mesh = plsc.VectorSubcoreMesh(core_axis_name="c", subcore_axis_name="s")

    @functools.partial(
        pl.kernel,
        mesh=mesh,
        out_type=jax.ShapeDtypeStruct((B, 1, CTX_DIM), token_embedding.dtype),
        scratch_types=[
            pltpu.VMEM((SUB0,), jnp.int32),
            pltpu.VMEM((SUB1,), jnp.int32),
            pltpu.VMEM((SUB0, 1, CTX_DIM), jnp.float32),
            pltpu.VMEM((SUB1, 1, CTX_DIM), jnp.float32),
            pltpu.SemaphoreType.DMA,
            pltpu.SemaphoreType.DMA,
        ],
    )
    def gather_kernel(table_hbm, idx_hbm, out_hbm,
                      idx0, idx1, rows0, rows1, sem0, sem1):
        wid = lax.axis_index("s") * NUM_CORES + lax.axis_index("c")
        base = wid * CHUNK

        @pl.when(wid < NW - 1)
        def _():
            pltpu.sync_copy(idx_hbm.at[pl.ds(base, SUB0)], idx0)
            pltpu.sync_copy(idx_hbm.at[pl.ds(base + SUB0, SUB1)], idx1)
            cp0 = pltpu.make_async_copy(table_hbm.at[idx0], rows0, sem0)
            cp1 = pltpu.make_async_copy(table_hbm.at[idx1], rows1, sem1)
            cp0.start()
            cp1.start()
            cp0.wait()
            pltpu.sync_copy(rows0, out_hbm.at[pl.ds(base, SUB0)])
            cp1.wait()
            pltpu.sync_copy(rows1, out_hbm.at[pl.ds(base + SUB0, SUB1)])

        @pl.when(wid == NW - 1)
        def _():
            pltpu.sync_copy(idx_hbm.at[pl.ds(base, SUB0)], idx0)
            cp = pltpu.make_async_copy(table_hbm.at[idx0], rows0, sem0)
            cp.start()
            cp.wait()
            pltpu.sync_copy(rows0.at[pl.ds(0, TAIL)],
                            out_hbm.at[pl.ds(base, TAIL)])

    out = gather_kernel(table, idx)
    return out.reshape(N_CLS, CTX_LEN, CTX_DIM)


# speedup vs baseline: 4.5124x; 4.5124x over previous
import functools

import jax
import jax.numpy as jnp
from jax import lax
from jax.experimental import pallas as pl
from jax.experimental.pallas import tpu as pltpu
from jax.experimental.pallas import tpu_sc as plsc

N_CLS = 101
CTX_LEN = 77
CTX_DIM = 512
B = N_CLS * CTX_LEN
NUM_CORES = 2
NUM_SUBCORES = 16
NW = NUM_CORES * NUM_SUBCORES
SUB0 = 128
SUB1 = 120
CHUNK = SUB0 + SUB1
TAIL = 120
TAIL_START = B - TAIL
STAGE0 = ((B + 7) // 8) * 8
STAGE1 = STAGE0 + TAIL


def kernel(tokenized_prompts, token_embedding):
    idx = tokenized_prompts.reshape(-1)
    idx_aug = jnp.concatenate([
        idx,
        jnp.zeros((STAGE0 - B,), jnp.int32),
        idx[TAIL_START:],
        jnp.arange(TAIL_START, B, dtype=jnp.int32),
    ])

    mesh = plsc.VectorSubcoreMesh(core_axis_name="c", subcore_axis_name="s")

    @functools.partial(
        pl.kernel,
        mesh=mesh,
        out_type=jax.ShapeDtypeStruct((B, CTX_DIM), token_embedding.dtype),
        scratch_types=[
            pltpu.VMEM((SUB0,), jnp.int32),
            pltpu.VMEM((SUB1,), jnp.int32),
            pltpu.VMEM((TAIL,), jnp.int32),
            pltpu.VMEM((SUB0, CTX_DIM), jnp.float32),
            pltpu.VMEM((SUB1, CTX_DIM), jnp.float32),
            pltpu.SemaphoreType.DMA,
            pltpu.SemaphoreType.DMA,
        ],
    )
    def gather_kernel(table_hbm, idx_hbm, out_hbm,
                      idx0, idx1, rowidx, rows0, rows1, sem0, sem1):
        wid = lax.axis_index("s") * NUM_CORES + lax.axis_index("c")
        base = wid * CHUNK

        @pl.when(wid < NW - 1)
        def _():
            pltpu.sync_copy(idx_hbm.at[pl.ds(base, SUB0)], idx0)
            pltpu.sync_copy(idx_hbm.at[pl.ds(base + SUB0, SUB1)], idx1)
            cp0 = pltpu.make_async_copy(table_hbm.at[idx0], rows0, sem0)
            cp1 = pltpu.make_async_copy(table_hbm.at[idx1], rows1, sem1)
            cp0.start()
            cp1.start()
            cp0.wait()
            pltpu.sync_copy(rows0, out_hbm.at[pl.ds(base, SUB0)])
            cp1.wait()
            pltpu.sync_copy(rows1, out_hbm.at[pl.ds(base + SUB0, SUB1)])

        @pl.when(wid == NW - 1)
        def _():
            pltpu.sync_copy(idx_hbm.at[pl.ds(STAGE0, TAIL)], idx1)
            pltpu.sync_copy(idx_hbm.at[pl.ds(STAGE1, TAIL)], rowidx)
            cp = pltpu.make_async_copy(table_hbm.at[idx1], rows1, sem1)
            cp.start()
            cp.wait()
            pltpu.sync_copy(rows1, out_hbm.at[rowidx])

    out = gather_kernel(token_embedding, idx_aug)
    return out.reshape(N_CLS, CTX_LEN, CTX_DIM)


# speedup vs baseline: 4.7117x; 1.0442x over previous
import functools

import jax
import jax.numpy as jnp
from jax import lax
from jax.experimental import pallas as pl
from jax.experimental.pallas import tpu as pltpu
from jax.experimental.pallas import tpu_sc as plsc

N_CLS = 101
CTX_LEN = 77
CTX_PAD = 80
CTX_ALN = 72
CTX_DIM = 512
LANES = 16
NUM_CORES = 2
NUM_SUBCORES = 16
NW = NUM_CORES * NUM_SUBCORES


def kernel(tokenized_prompts, token_embedding):
    idx_flat = tokenized_prompts.reshape(-1)
    pos = jnp.arange(N_CLS)[:, None] * CTX_LEN + jnp.arange(CTX_PAD)[None, :]
    idxp = idx_flat[jnp.minimum(pos, N_CLS * CTX_LEN - 1)].reshape(-1)

    mesh = plsc.VectorSubcoreMesh(core_axis_name="c", subcore_axis_name="s")

    @functools.partial(
        pl.kernel,
        mesh=mesh,
        out_type=jax.ShapeDtypeStruct(
            (N_CLS, CTX_LEN, CTX_DIM), token_embedding.dtype),
        scratch_types=[
            pltpu.VMEM((CTX_PAD,), jnp.int32),
            pltpu.VMEM((CTX_PAD,), jnp.int32),
            pltpu.VMEM((CTX_LEN, CTX_DIM), jnp.float32),
            pltpu.VMEM((CTX_LEN, CTX_DIM), jnp.float32),
            pltpu.VMEM((8, CTX_DIM), jnp.float32),
            pltpu.VMEM((8, CTX_DIM), jnp.float32),
            pltpu.SemaphoreType.DMA,
            pltpu.SemaphoreType.DMA,
            pltpu.SemaphoreType.DMA,
            pltpu.SemaphoreType.DMA,
        ],
    )
    def gather_kernel(table_hbm, idx_hbm, out_hbm,
                      i0, i1, ma, mb, ta, tb, s0, s1, s2, s3):
        wid = lax.axis_index("s") * NUM_CORES + lax.axis_index("c")
        c0 = 3 * wid + jnp.minimum(wid, 5)
        ibufs = (i0, i1)
        mains = (ma, mb)
        tails = (ta, tb)
        msems = (s0, s1)
        tsems = (s2, s3)

        def start(k):
            b = k % 2
            pltpu.sync_copy(
                idx_hbm.at[pl.ds(CTX_PAD * (c0 + k), CTX_PAD)], ibufs[b])
            main_cp = pltpu.make_async_copy(
                table_hbm.at[ibufs[b].at[pl.ds(0, CTX_ALN)]],
                mains[b].at[pl.ds(0, CTX_ALN)], msems[b])
            tail_cp = pltpu.make_async_copy(
                table_hbm.at[ibufs[b].at[pl.ds(CTX_ALN, 8)]],
                tails[b], tsems[b])
            main_cp.start()
            tail_cp.start()
            return main_cp, tail_cp

        def finish(k, cps):
            b = k % 2
            main_cp, tail_cp = cps
            main_cp.wait()
            tail_cp.wait()
            for r in range(CTX_LEN - CTX_ALN):
                for j in range(0, CTX_DIM, LANES):
                    mains[b][pl.ds(CTX_ALN + r, 1), pl.ds(j, LANES)] = (
                        tails[b][pl.ds(r, 1), pl.ds(j, LANES)])
            pltpu.sync_copy(mains[b], out_hbm.at[c0 + k])

        def run(nc):
            cps = start(0)
            for k in range(nc):
                nxt = start(k + 1) if k + 1 < nc else None
                finish(k, cps)
                cps = nxt

        @pl.when(wid < 5)
        def _():
            run(4)

        @pl.when(wid >= 5)
        def _():
            run(3)

    return gather_kernel(token_embedding, idxp)


# speedup vs baseline: 8.2487x; 1.7507x over previous
import functools

import jax
import jax.numpy as jnp
from jax import lax
from jax.experimental import pallas as pl
from jax.experimental.pallas import tpu as pltpu
from jax.experimental.pallas import tpu_sc as plsc

N_CLS = 101
CTX_LEN = 77
CLS_PAD = 104
CLS_ALN = 96
CTX_DIM = 512
LANES = 16
NUM_CORES = 2
NUM_SUBCORES = 16
NW = NUM_CORES * NUM_SUBCORES


def kernel(tokenized_prompts, token_embedding):
    idx_t = tokenized_prompts.T
    idxp = jnp.concatenate([idx_t, idx_t[:, -3:]], axis=1).reshape(-1)

    mesh = plsc.VectorSubcoreMesh(core_axis_name="c", subcore_axis_name="s")

    @functools.partial(
        pl.kernel,
        mesh=mesh,
        out_type=jax.ShapeDtypeStruct(
            (CTX_LEN, N_CLS, CTX_DIM), token_embedding.dtype),
        scratch_types=[
            pltpu.VMEM((CLS_PAD,), jnp.int32),
            pltpu.VMEM((CLS_PAD,), jnp.int32),
            pltpu.VMEM((N_CLS, CTX_DIM), jnp.float32),
            pltpu.VMEM((N_CLS, CTX_DIM), jnp.float32),
            pltpu.VMEM((8, CTX_DIM), jnp.float32),
            pltpu.VMEM((8, CTX_DIM), jnp.float32),
            pltpu.SemaphoreType.DMA,
            pltpu.SemaphoreType.DMA,
            pltpu.SemaphoreType.DMA,
            pltpu.SemaphoreType.DMA,
        ],
    )
    def gather_kernel(table_hbm, idx_hbm, out_hbm,
                      i0, i1, ma, mb, ta, tb, s0, s1, s2, s3):
        wid = lax.axis_index("s") * NUM_CORES + lax.axis_index("c")
        t0 = 2 * wid + jnp.minimum(wid, 13)
        ibufs = (i0, i1)
        mains = (ma, mb)
        tails = (ta, tb)
        msems = (s0, s1)
        tsems = (s2, s3)

        def start(k):
            b = k % 2
            pltpu.sync_copy(
                idx_hbm.at[pl.ds(CLS_PAD * (t0 + k), CLS_PAD)], ibufs[b])
            main_cp = pltpu.make_async_copy(
                table_hbm.at[ibufs[b].at[pl.ds(0, CLS_ALN)]],
                mains[b].at[pl.ds(0, CLS_ALN)], msems[b])
            tail_cp = pltpu.make_async_copy(
                table_hbm.at[ibufs[b].at[pl.ds(CLS_ALN, 8)]],
                tails[b], tsems[b])
            main_cp.start()
            tail_cp.start()
            return main_cp, tail_cp

        def finish(k, cps):
            b = k % 2
            main_cp, tail_cp = cps
            main_cp.wait()
            tail_cp.wait()
            for r in range(N_CLS - CLS_ALN):
                for j in range(0, CTX_DIM, LANES):
                    mains[b][pl.ds(CLS_ALN + r, 1), pl.ds(j, LANES)] = (
                        tails[b][pl.ds(r, 1), pl.ds(j, LANES)])
            pltpu.sync_copy(mains[b], out_hbm.at[t0 + k])

        def run(nt):
            cps = start(0)
            for k in range(nt):
                nxt = start(k + 1) if k + 1 < nt else None
                finish(k, cps)
                cps = nxt

        @pl.when(wid < 13)
        def _():
            run(3)

        @pl.when(wid >= 13)
        def _():
            run(2)

    out_t = gather_kernel(token_embedding, idxp)
    return jnp.transpose(out_t, (1, 0, 2))


# speedup vs baseline: 8.6659x; 1.0506x over previous
import functools

import jax
import jax.numpy as jnp
from jax import lax
from jax.experimental import pallas as pl
from jax.experimental.pallas import tpu as pltpu
from jax.experimental.pallas import tpu_sc as plsc

N_CLS = 101
CTX_LEN = 77
CLS_PAD = 104
CTX_DIM = 512
NUM_CORES = 2
NUM_SUBCORES = 16
NW = NUM_CORES * NUM_SUBCORES


def kernel(tokenized_prompts, token_embedding):
    idx_t = tokenized_prompts.T
    idxp = jnp.concatenate([idx_t, idx_t[:, -3:]], axis=1).reshape(-1)

    mesh = plsc.VectorSubcoreMesh(core_axis_name="c", subcore_axis_name="s")

    @functools.partial(
        pl.kernel,
        mesh=mesh,
        out_type=jax.ShapeDtypeStruct(
            (CTX_LEN, CLS_PAD, CTX_DIM), token_embedding.dtype),
        scratch_types=[
            pltpu.VMEM((CLS_PAD,), jnp.int32),
            pltpu.VMEM((CLS_PAD,), jnp.int32),
            pltpu.VMEM((CLS_PAD, CTX_DIM), jnp.float32),
            pltpu.VMEM((CLS_PAD, CTX_DIM), jnp.float32),
            pltpu.SemaphoreType.DMA,
            pltpu.SemaphoreType.DMA,
        ],
    )
    def gather_kernel(table_hbm, idx_hbm, out_hbm,
                      i0, i1, ma, mb, s0, s1):
        wid = lax.axis_index("s") * NUM_CORES + lax.axis_index("c")
        t0 = 2 * wid + jnp.minimum(wid, 13)
        ibufs = (i0, i1)
        mains = (ma, mb)
        msems = (s0, s1)

        def start(k):
            b = k % 2
            pltpu.sync_copy(
                idx_hbm.at[pl.ds(CLS_PAD * (t0 + k), CLS_PAD)], ibufs[b])
            cp = pltpu.make_async_copy(
                table_hbm.at[ibufs[b]], mains[b], msems[b])
            cp.start()
            return cp

        def finish(k, cp):
            cp.wait()
            pltpu.sync_copy(mains[k % 2], out_hbm.at[t0 + k])

        def run(nt):
            cp = start(0)
            for k in range(nt):
                nxt = start(k + 1) if k + 1 < nt else None
                finish(k, cp)
                cp = nxt

        @pl.when(wid < 13)
        def _():
            run(3)

        @pl.when(wid >= 13)
        def _():
            run(2)

    out_t = gather_kernel(token_embedding, idxp)
    return jnp.transpose(out_t, (1, 0, 2))[:N_CLS]
